# baseline (speedup 1.0000x reference)
import functools

import numpy as np
import jax
import jax.numpy as jnp
from jax import lax
from jax.experimental import pallas as pl
from jax.experimental.pallas import tpu as pltpu
from jax.experimental.pallas import tpu_sc as plsc

_LENGTH = 200
_D = 64
_BATCH = 4096
_V = 1_000_000
_NC = 2
_NS = 16
_NW = _NC * _NS
_BW = _BATCH // _NW
_NB = 4
_AHEAD = 2
_SCALE = 8.0

_NBLK = _V // 128
_TAIL = _V - _NBLK * 128
_TAIL_W = _NBLK % _NW


def _pos_enc() -> np.ndarray:
    depth = _D / 2
    positions = np.arange(_LENGTH)[:, None]
    depths = np.arange(depth)[None, :] / depth
    angle_rates = 1 / 10000 ** depths
    angle_rads = positions * angle_rates
    pe = np.concatenate([np.sin(angle_rads), np.cos(angle_rads)], axis=-1)
    return pe.astype(np.float32)


_mesh = plsc.VectorSubcoreMesh(core_axis_name="c", subcore_axis_name="s")
_compact = pltpu.CompilerParams(
    use_tc_tiling_on_sc=True, needs_layout_passes=False)


@functools.partial(
    pl.kernel,
    mesh=_mesh,
    out_type=jax.ShapeDtypeStruct((_V // 2, 128), jnp.float32),
    scratch_types=[
        pltpu.VMEM((2, _D, 128), jnp.float32),
        pltpu.VMEM((2, _D, 128), jnp.float32),
        pltpu.SemaphoreType.DMA,
        pltpu.SemaphoreType.DMA,
        pltpu.SemaphoreType.DMA,
        pltpu.SemaphoreType.DMA,
    ],
    compiler_params=_compact,
)
def _sc_transpose(tt_hbm, tail_hbm, t2_hbm, in_v, out_v,
                  isem0, isem1, osem0, osem1):
    isem = (isem0, isem1)
    osem = (osem0, osem1)
    wid = lax.axis_index("s") * _NC + lax.axis_index("c")
    nblk = 244 + jnp.where(wid < _NBLK % _NW, 1, 0)
    iota = lax.iota(jnp.int32, 16)
    fids = [16 * q + iota for q in range(_D // 16)]

    def in_copy(k, b):
        j = lax.min(wid + k * _NW, _NBLK - 1)
        return pltpu.make_async_copy(
            tt_hbm.at[:, pl.ds(pl.multiple_of(j * 128, 128), 128)],
            in_v.at[b], isem[b])

    def out_copy(k, b):
        j = lax.min(wid + k * _NW, _NBLK - 1)
        return pltpu.make_async_copy(
            out_v.at[b],
            t2_hbm.at[pl.ds(pl.multiple_of(j * 64, 64), _D), :], osem[b])

    def transpose_block(b, nrows=_D):
        def row_body(r2, _):
            for half in range(2):
                r = jnp.full((16,), 2 * r2 + half, jnp.int32)
                for q in range(_D // 16):
                    v = plsc.load_gather(in_v.at[b], [fids[q], r])
                    out_v[b, r2, pl.ds(64 * half + 16 * q, 16)] = v * _SCALE
            return 0
        lax.fori_loop(0, nrows, row_body, 0, unroll=2)

    in_copy(0, 0).start()

    def step(k, _):
        b = lax.rem(k, 2)
        for bb in range(2):
            @pl.when(b == bb)
            def _():
                @pl.when(k + 1 < nblk)
                def _():
                    in_copy(k + 1, 1 - bb).start()
                in_copy(k, bb).wait()

                @pl.when(k >= 2)
                def _():
                    out_copy(k - 2, bb).wait()
                transpose_block(bb)
                out_copy(k, bb).start()
        return 0

    lax.fori_loop(0, nblk, step, 0)

    par = lax.rem(nblk, 2)
    for p in range(2):
        @pl.when(par == p)
        def _():
            out_copy(nblk - 2, p).wait()
            out_copy(nblk - 1, 1 - p).wait()

    @pl.when(wid == _TAIL_W)
    def _():
        pltpu.sync_copy(tail_hbm, in_v.at[0, pl.ds(0, _TAIL // 2), :])
        pltpu.sync_copy(in_v.at[0, pl.ds(0, _TAIL // 2), :],
                        t2_hbm.at[pl.ds(_V // 2 - _TAIL // 2, _TAIL // 2), :])


@functools.partial(
    pl.kernel,
    mesh=_mesh,
    out_type=jax.ShapeDtypeStruct((_LENGTH, _D, _BATCH), jnp.float32),
    scratch_types=[
        pltpu.VMEM((_LENGTH, _BW), jnp.int32),
        pltpu.VMEM((8, _BW), jnp.int32),
        pltpu.VMEM((2, 16 * _D), jnp.float32),
        pltpu.VMEM((_NB, _BW, 128), jnp.float32),
        pltpu.VMEM((2, _D, _BW), jnp.float32),
    ] + [pltpu.SemaphoreType.DMA] * (_NB + 4),
    compiler_params=_compact,
)
def _sc_embed2(t2_hbm, xt_hbm, pe_hbm, out_hbm,
               idx_v, pair_v, peb_v, gath_v, out_v, *sems):
    gsem, osem, psem = sems[:_NB], sems[_NB:_NB + 2], sems[_NB + 2:]
    wid = lax.axis_index("s") * _NC + lax.axis_index("c")
    b0 = pl.multiple_of(wid * _BW, _BW)
    pltpu.sync_copy(xt_hbm.at[:, pl.ds(b0, _BW)], idx_v)
    iota = lax.iota(jnp.int32, 16)
    rowids = [16 * m + iota for m in range(_BW // 16)]

    def fill_pairs(t, b):
        for m in range(_BW // 16):
            iv = idx_v[t, pl.ds(16 * m, 16)]
            pair_v[b, pl.ds(16 * m, 16)] = lax.shift_right_logical(iv, 1)

    def gather(b):
        return pltpu.make_async_copy(
            t2_hbm.at[pair_v.at[b]], gath_v.at[b], gsem[b])

    def pe_copy(t, pb):
        t = lax.min(t, _LENGTH - 1)
        return pltpu.make_async_copy(pe_hbm.at[t], peb_v.at[pb], psem[pb])

    def out_copy(t, ob):
        t = lax.max(t, 0)
        return pltpu.make_async_copy(
            out_v.at[ob], out_hbm.at[t, :, pl.ds(b0, _BW)], osem[ob])

    for b in range(_AHEAD):
        fill_pairs(b, b)
        gather(b).start()
    pe_copy(0, 0).start()
    pe_copy(1, 1).start()

    def substep(t, b, ob, pb):
        bn = (b + _AHEAD) % _NB

        @pl.when(t + _AHEAD < _LENGTH)
        def _():
            fill_pairs(t + _AHEAD, bn)
            gather(bn).start()

        gather(b).wait()
        pe_copy(t, pb).wait()

        @pl.when(t >= 2)
        def _():
            out_copy(t - 2, ob).wait()

        hv64 = []
        for m in range(_BW // 16):
            iv = idx_v[t, pl.ds(16 * m, 16)]
            hv64.append(lax.shift_left(jnp.bitwise_and(iv, 1), 6))

        def f_body(f, _):
            pv = peb_v[pb, pl.ds(16 * f, 16)]
            for m in range(_BW // 16):
                col = hv64[m] + f
                v = plsc.load_gather(gath_v.at[b], [rowids[m], col])
                out_v[ob, f, pl.ds(16 * m, 16)] = v + pv
            return 0

        lax.fori_loop(0, _D, f_body, 0, unroll=2)

        @pl.when(t + 2 < _LENGTH)
        def _():
            pe_copy(t + 2, pb).start()
        out_copy(t, ob).start()

    def outer(i, _):
        for bb in range(_NB):
            t = i * _NB + bb
            substep(t, bb, bb % 2, bb % 2)
        return 0

    lax.fori_loop(0, _LENGTH // _NB, outer, 0)
    out_copy(_LENGTH - 2, 0).wait()
    out_copy(_LENGTH - 1, 1).wait()


def kernel(x, embedding_table):
    tt = embedding_table.T
    xt = x.T.astype(jnp.int32)
    pe = jnp.asarray(np.repeat(_pos_enc(), 16, axis=1))
    tail = (embedding_table[_NBLK * 128:] * _SCALE).reshape(_TAIL // 2, 128)
    t2 = _sc_transpose(tt, tail)
    out3 = _sc_embed2(t2, xt, pe)
    return jnp.transpose(out3, (2, 0, 1))


# speedup vs baseline: 1.2348x; 1.2348x over previous
import functools

import numpy as np
import jax
import jax.numpy as jnp
from jax import lax
from jax.experimental import pallas as pl
from jax.experimental.pallas import tpu as pltpu
from jax.experimental.pallas import tpu_sc as plsc

_LENGTH = 200
_D = 64
_BATCH = 4096
_V = 1_000_000
_NC = 2
_NS = 16
_NW = _NC * _NS
_BW = _BATCH // _NW
_NB = 2
_AHEAD = 1
_SCALE = 8.0
_P = 129

_NBLK = _V // 128
_TAIL = _V - _NBLK * 128
_TAIL_W = _NBLK % _NW


def _pos_enc() -> np.ndarray:
    depth = _D / 2
    positions = np.arange(_LENGTH)[:, None]
    depths = np.arange(depth)[None, :] / depth
    angle_rates = 1 / 10000 ** depths
    angle_rads = positions * angle_rates
    pe = np.concatenate([np.sin(angle_rads), np.cos(angle_rads)], axis=-1)
    return pe.astype(np.float32)


_mesh = plsc.VectorSubcoreMesh(core_axis_name="c", subcore_axis_name="s")
_compact = pltpu.CompilerParams(
    use_tc_tiling_on_sc=True, needs_layout_passes=False)


@functools.partial(
    pl.kernel,
    mesh=_mesh,
    out_type=jax.ShapeDtypeStruct((_V // 2, 128), jnp.float32),
    scratch_types=[
        pltpu.VMEM((2, _D, 128), jnp.float32),
        pltpu.VMEM((2, _D, 128), jnp.float32),
        pltpu.VMEM((_D * _P,), jnp.float32),
        pltpu.SemaphoreType.DMA,
        pltpu.SemaphoreType.DMA,
        pltpu.SemaphoreType.DMA,
        pltpu.SemaphoreType.DMA,
    ],
    compiler_params=_compact,
)
def _sc_transpose(tt_hbm, tail_hbm, t2_hbm, in_v, out_v, st_v,
                  isem0, isem1, osem0, osem1):
    isem = (isem0, isem1)
    osem = (osem0, osem1)
    wid = lax.axis_index("s") * _NC + lax.axis_index("c")
    nblk = 244 + jnp.where(wid < _NBLK % _NW, 1, 0)
    iota = lax.iota(jnp.int32, 16)
    fqp = [(16 * q + iota) * _P for q in range(_D // 16)]

    def in_copy(k, b):
        j = lax.min(wid + k * _NW, _NBLK - 1)
        return pltpu.make_async_copy(
            tt_hbm.at[:, pl.ds(pl.multiple_of(j * 128, 128), 128)],
            in_v.at[b], isem[b])

    def out_copy(k, b):
        j = lax.min(wid + k * _NW, _NBLK - 1)
        return pltpu.make_async_copy(
            out_v.at[b],
            t2_hbm.at[pl.ds(pl.multiple_of(j * 64, 64), _D), :], osem[b])

    def transpose_block(b):
        def f_body(f, _):
            base = f * _P
            for k in range(8):
                v = in_v[b, f, pl.ds(16 * k, 16)]
                st_v[pl.ds(base + 16 * k, 16)] = v * _SCALE
            return 0

        lax.fori_loop(0, _D, f_body, 0, unroll=2)

        def r_body(r2, _):
            for k in range(8):
                idx = fqp[k % 4] + (2 * r2 + k // 4)
                v = plsc.load_gather(st_v, [idx])
                out_v[b, r2, pl.ds(16 * k, 16)] = v
            return 0

        lax.fori_loop(0, _D, r_body, 0, unroll=2)

    in_copy(0, 0).start()

    def step(k, _):
        b = lax.rem(k, 2)
        for bb in range(2):
            @pl.when(b == bb)
            def _():
                @pl.when(k + 1 < nblk)
                def _():
                    in_copy(k + 1, 1 - bb).start()
                in_copy(k, bb).wait()

                @pl.when(k >= 2)
                def _():
                    out_copy(k - 2, bb).wait()
                transpose_block(bb)
                out_copy(k, bb).start()
        return 0

    lax.fori_loop(0, nblk, step, 0)

    par = lax.rem(nblk, 2)
    for p in range(2):
        @pl.when(par == p)
        def _():
            out_copy(nblk - 2, p).wait()
            out_copy(nblk - 1, 1 - p).wait()

    @pl.when(wid == _TAIL_W)
    def _():
        pltpu.sync_copy(tail_hbm, in_v.at[0, pl.ds(0, _TAIL // 2), :])
        pltpu.sync_copy(in_v.at[0, pl.ds(0, _TAIL // 2), :],
                        t2_hbm.at[pl.ds(_V // 2 - _TAIL // 2, _TAIL // 2), :])


@functools.partial(
    pl.kernel,
    mesh=_mesh,
    out_type=jax.ShapeDtypeStruct((_LENGTH, _D, _BATCH), jnp.float32),
    scratch_types=[
        pltpu.VMEM((_LENGTH, _BW), jnp.int32),
        pltpu.VMEM((8, _BW), jnp.int32),
        pltpu.VMEM((100, 128), jnp.float32),
        pltpu.VMEM((_NB, _BW, 128), jnp.float32),
        pltpu.VMEM((_BW * _P,), jnp.float32),
        pltpu.VMEM((2, _D, _BW), jnp.float32),
    ] + [pltpu.SemaphoreType.DMA] * (_NB + 2),
    compiler_params=_compact,
)
def _sc_embed2(t2_hbm, xt_hbm, pe_hbm, out_hbm,
               idx_v, pair_v, pe_v, gath_v, st_v, out_v, *sems):
    gsem, osem = sems[:_NB], sems[_NB:]
    wid = lax.axis_index("s") * _NC + lax.axis_index("c")
    b0 = pl.multiple_of(wid * _BW, _BW)
    pltpu.sync_copy(xt_hbm.at[:, pl.ds(b0, _BW)], idx_v)
    pltpu.sync_copy(pe_hbm, pe_v)
    iota = lax.iota(jnp.int32, 16)
    rqp = [(16 * m + iota) * _P for m in range(_BW // 16)]

    def fill_pairs(t, b):
        for m in range(_BW // 16):
            iv = idx_v[t, pl.ds(16 * m, 16)]
            pair_v[b, pl.ds(16 * m, 16)] = lax.shift_right_logical(iv, 1)

    def gather(b):
        return pltpu.make_async_copy(
            t2_hbm.at[pair_v.at[b]], gath_v.at[b], gsem[b])

    def out_copy(t, ob):
        t = lax.max(t, 0)
        return pltpu.make_async_copy(
            out_v.at[ob], out_hbm.at[t, :, pl.ds(b0, _BW)], osem[ob])

    fill_pairs(0, 0)
    gather(0).start()

    def substep(t, b):
        ob = b

        @pl.when(t + 1 < _LENGTH)
        def _():
            fill_pairs(t + 1, 1 - b)
            gather(1 - b).start()

        gather(b).wait()

        @pl.when(t >= 2)
        def _():
            out_copy(t - 2, ob).wait()

        thalf = lax.rem(t, 2)
        trow = lax.div(t, 2)
        pq = tuple(pe_v[trow, pl.ds(thalf * _D + 16 * q, 16)]
                   for q in range(_D // 16))

        def b_body(r, carry):
            base = r * _P
            for k in range(8):
                v = gath_v[b, r, pl.ds(16 * k, 16)]
                st_v[pl.ds(base + 16 * k, 16)] = v + carry[k % 4]
            return carry

        lax.fori_loop(0, _BW, b_body, pq, unroll=2)

        hv64 = []
        for m in range(_BW // 16):
            iv = idx_v[t, pl.ds(16 * m, 16)]
            hv64.append(lax.shift_left(jnp.bitwise_and(iv, 1), 6))

        def f_body(f, _):
            for m in range(_BW // 16):
                idx = rqp[m] + hv64[m] + f
                v = plsc.load_gather(st_v, [idx])
                out_v[ob, f, pl.ds(16 * m, 16)] = v
            return 0

        lax.fori_loop(0, _D, f_body, 0, unroll=2)
        out_copy(t, ob).start()

    def outer(i, _):
        for bb in range(2):
            substep(i * 2 + bb, bb)
        return 0

    lax.fori_loop(0, _LENGTH // 2, outer, 0)
    out_copy(_LENGTH - 2, 0).wait()
    out_copy(_LENGTH - 1, 1).wait()


def kernel(x, embedding_table):
    tt = embedding_table.T
    xt = x.T.astype(jnp.int32)
    pe = jnp.asarray(_pos_enc().reshape(100, 128))
    tail = (embedding_table[_NBLK * 128:] * _SCALE).reshape(_TAIL // 2, 128)
    t2 = _sc_transpose(tt, tail)
    out3 = _sc_embed2(t2, xt, pe)
    return jnp.transpose(out3, (2, 0, 1))
